# Initial kernel scaffold; baseline (speedup 1.0000x reference)
import jax
import jax.numpy as jnp
from jax.experimental import pallas as pl


def kernel(x, edge_index, W1, b1, W2, b2):
    raise NotImplementedError("write your pallas kernel here")


# speedup vs baseline: 13.7722x; 13.7722x over previous
import functools

import jax
import jax.numpy as jnp
from jax import lax
from jax.experimental import pallas as pl
from jax.experimental.pallas import tpu as pltpu
from jax.experimental.pallas import tpu_sc as plsc

N = 10000
E = 320000
IN_CH = 128
HID_CH = 256
OUT_CH = 128

NCORES = 2
NSUB = 16
NW = NCORES * NSUB
CHUNK = 128
NCH = 79
E_PAD = NW * NCH * CHUNK
PAD_ROW = N
NROWS = 10112
RPT = NROWS // NSUB

_MESH = plsc.VectorSubcoreMesh(core_axis_name="c", subcore_axis_name="s")


@functools.partial(
    pl.kernel,
    mesh=_MESH,
    out_type=jax.ShapeDtypeStruct((NCORES, NROWS, IN_CH), jnp.float32),
    scratch_types=[
        pltpu.VMEM((NCH, CHUNK), jnp.int32),
        pltpu.VMEM((CHUNK, IN_CH), jnp.float32),
        pltpu.VMEM_SHARED((NROWS, IN_CH), jnp.float32),
    ],
)
def _deg_kernel(dst_hbm, ones_hbm, zeros_hbm, out_hbm, dst_v, ones_v, acc):
    c = lax.axis_index("c")
    s = lax.axis_index("s")
    pltpu.sync_copy(zeros_hbm.at[pl.ds(s * RPT, RPT)], acc.at[pl.ds(s * RPT, RPT)])
    pltpu.sync_copy(ones_hbm, ones_v)
    g = c * NSUB + s
    pltpu.sync_copy(dst_hbm.at[g], dst_v)
    plsc.subcore_barrier()

    def body(j, carry):
        pltpu.sync_copy(ones_v, acc.at[dst_v.at[j]], add=True)
        return carry

    lax.fori_loop(0, NCH, body, 0)
    plsc.subcore_barrier()
    pltpu.sync_copy(acc.at[pl.ds(s * RPT, RPT)], out_hbm.at[c, pl.ds(s * RPT, RPT)])


@functools.partial(
    pl.kernel,
    mesh=_MESH,
    out_type=jax.ShapeDtypeStruct((NCORES, NROWS, IN_CH), jnp.float32),
    scratch_types=[
        pltpu.VMEM((NCH, CHUNK), jnp.int32),
        pltpu.VMEM((NCH, CHUNK), jnp.int32),
        pltpu.VMEM((CHUNK, IN_CH), jnp.float32),
        pltpu.VMEM_SHARED((NROWS, IN_CH), jnp.float32),
        pltpu.SemaphoreType.DMA,
    ],
)
def _agg_kernel(src_hbm, dst_hbm, table_hbm, zeros_hbm, out_hbm,
                src_v, dst_v, rows_v, acc, sem):
    c = lax.axis_index("c")
    s = lax.axis_index("s")
    pltpu.sync_copy(zeros_hbm.at[pl.ds(s * RPT, RPT)], acc.at[pl.ds(s * RPT, RPT)])
    g = c * NSUB + s
    pltpu.sync_copy(src_hbm.at[g], src_v)
    pltpu.sync_copy(dst_hbm.at[g], dst_v)
    plsc.subcore_barrier()

    def body(j, carry):
        pltpu.async_copy(table_hbm.at[src_v.at[j]], rows_v, sem).wait()
        pltpu.sync_copy(rows_v, acc.at[dst_v.at[j]], add=True)
        return carry

    lax.fori_loop(0, NCH, body, 0)
    plsc.subcore_barrier()
    pltpu.sync_copy(acc.at[pl.ds(s * RPT, RPT)], out_hbm.at[c, pl.ds(s * RPT, RPT)])


_R = 1000


def _prep_body(d0_ref, d1_ref, x_ref, dinv_ref, xp_ref):
    deg = d0_ref[...] + d1_ref[...] + 1.0
    dv = lax.rsqrt(deg)
    dinv_ref[...] = dv
    xp_ref[...] = dv * x_ref[...]


def _mid_body(p0_ref, p1_ref, xp_ref, dinv_ref, w1_ref, b1_ref, w2_ref, zp_ref):
    dv = dinv_ref[...]
    y = dv * (p0_ref[...] + p1_ref[...] + xp_ref[...])
    h = jnp.dot(y, w1_ref[...], preferred_element_type=jnp.float32,
                precision=lax.Precision.HIGHEST) + b1_ref[...]
    h = jnp.maximum(h, 0.0)
    z = jnp.dot(h, w2_ref[...], preferred_element_type=jnp.float32,
                precision=lax.Precision.HIGHEST)
    zp_ref[...] = dv * z


def _fin_body(q0_ref, q1_ref, zp_ref, dinv_ref, b2_ref, out_ref):
    out_ref[...] = dinv_ref[...] * (q0_ref[...] + q1_ref[...] + zp_ref[...]) + b2_ref[...]


def _rows(width):
    return pl.BlockSpec((_R, width), lambda i: (i, 0))


def _full(a, b):
    return pl.BlockSpec((a, b), lambda i: (0, 0))


_prep = pl.pallas_call(
    _prep_body,
    grid=(N // _R,),
    in_specs=[_rows(1), _rows(1), _rows(IN_CH)],
    out_specs=[_rows(1), _rows(IN_CH)],
    out_shape=[jax.ShapeDtypeStruct((N, 1), jnp.float32),
               jax.ShapeDtypeStruct((N, IN_CH), jnp.float32)],
)

_mid = pl.pallas_call(
    _mid_body,
    grid=(N // _R,),
    in_specs=[_rows(IN_CH), _rows(IN_CH), _rows(IN_CH), _rows(1),
              _full(IN_CH, HID_CH), _full(1, HID_CH), _full(HID_CH, OUT_CH)],
    out_specs=_rows(OUT_CH),
    out_shape=jax.ShapeDtypeStruct((N, OUT_CH), jnp.float32),
)

_fin = pl.pallas_call(
    _fin_body,
    grid=(N // _R,),
    in_specs=[_rows(OUT_CH), _rows(OUT_CH), _rows(OUT_CH), _rows(1),
              _full(1, OUT_CH)],
    out_specs=_rows(OUT_CH),
    out_shape=jax.ShapeDtypeStruct((N, OUT_CH), jnp.float32),
)


def kernel(x, edge_index, W1, b1, W2, b2):
    src = edge_index[0].astype(jnp.int32)
    dst = edge_index[1].astype(jnp.int32)
    pad = E_PAD - E
    src_b = jnp.concatenate([src, jnp.zeros((pad,), jnp.int32)]).reshape(NW, NCH, CHUNK)
    dst_b = jnp.concatenate([dst, jnp.full((pad,), PAD_ROW, jnp.int32)]).reshape(NW, NCH, CHUNK)

    ones128 = jnp.ones((CHUNK, IN_CH), jnp.float32)
    zeros128 = jnp.zeros((NROWS, IN_CH), jnp.float32)

    deg_p = _deg_kernel(dst_b, ones128, zeros128)
    d0 = deg_p[0, :N, 0:1]
    d1 = deg_p[1, :N, 0:1]
    dinv, xp = _prep(d0, d1, x)

    p = _agg_kernel(src_b, dst_b, xp, zeros128)
    zp = _mid(p[0, :N], p[1, :N], xp, dinv, W1, b1.reshape(1, HID_CH), W2)

    q = _agg_kernel(src_b, dst_b, zp, zeros128)
    out = _fin(q[0, :N], q[1, :N], zp, dinv, b2.reshape(1, OUT_CH))
    return out
